# Initial kernel scaffold; baseline (speedup 1.0000x reference)
import jax
import jax.numpy as jnp
from jax.experimental import pallas as pl


def kernel(x, edge_index, edge_attr, u, batch, W1, b1, W2, b2):
    raise NotImplementedError("write your pallas kernel here")


# speedup vs baseline: 3.1079x; 3.1079x over previous
import functools

import jax
import jax.numpy as jnp
from jax import lax
from jax.experimental import pallas as pl
from jax.experimental.pallas import tpu as pltpu
from jax.experimental.pallas import tpu_sc as plsc

N_ROWS = 100000
D = 128
NSEG = 256
NC = 2
NS = 16
NW = NC * NS
CHUNK = 160
NCHUNK = N_ROWS // CHUNK
MAXK = -(-NCHUNK // NW)
LANES = 16


def _sc_body(x_hbm, ids_hbm, sums_hbm, counts_hbm, xbuf, idsbuf, acc, cnt):
    cid = lax.axis_index("c")
    sid = lax.axis_index("s")
    wid = sid * NC + cid

    zeros16 = jnp.zeros((LANES,), jnp.float32)
    ones16 = jnp.ones((LANES,), jnp.float32)
    lane_iota = lax.iota(jnp.int32, LANES)

    def zero_row(r, carry):
        for c8 in range(D // LANES):
            acc[r, pl.ds(c8 * LANES, LANES)] = zeros16
        cnt[r, :] = zeros16
        return carry

    lax.fori_loop(0, NSEG, zero_row, 0)

    def chunk_body(k, carry):
        chunk_id = wid + NW * k

        @pl.when(chunk_id < NCHUNK)
        def _():
            base_row = chunk_id * CHUNK
            pltpu.sync_copy(x_hbm.at[pl.ds(base_row, CHUNK), :], xbuf)
            pltpu.sync_copy(ids_hbm.at[pl.ds(base_row, CHUNK)], idsbuf)

            def group_body(g, inner):
                gbase = g * LANES
                idv = idsbuf[pl.ds(gbase, LANES)]
                plsc.addupdate_scatter(cnt, [idv, lane_iota], ones16)
                for r in range(LANES):
                    seg = idv[r]
                    for c8 in range(D // LANES):
                        v = xbuf[gbase + r, pl.ds(c8 * LANES, LANES)]
                        plsc.addupdate(acc.at[seg, pl.ds(c8 * LANES, LANES)], v)
                return inner

            lax.fori_loop(0, CHUNK // LANES, group_body, 0)

        return carry

    lax.fori_loop(0, MAXK, chunk_body, 0)

    pltpu.sync_copy(acc, sums_hbm.at[wid])
    pltpu.sync_copy(cnt, counts_hbm.at[wid])


def _sc_segment_partials(x, ids):
    mesh = plsc.VectorSubcoreMesh(core_axis_name="c", subcore_axis_name="s")
    f = pl.kernel(
        _sc_body,
        out_type=(
            jax.ShapeDtypeStruct((NW, NSEG, D), jnp.float32),
            jax.ShapeDtypeStruct((NW, NSEG, LANES), jnp.float32),
        ),
        mesh=mesh,
        scratch_types=(
            pltpu.VMEM((CHUNK, D), jnp.float32),
            pltpu.VMEM((CHUNK,), jnp.int32),
            pltpu.VMEM((NSEG, D), jnp.float32),
            pltpu.VMEM((NSEG, LANES), jnp.float32),
        ),
        compiler_params=pltpu.CompilerParams(needs_layout_passes=False),
    )
    return f(x, ids)


def _tc_body(sums_ref, counts_ref, u_ref, W1_ref, b1_ref, W2_ref, b2_ref,
             out_ref):
    seg_sum = jnp.sum(sums_ref[...], axis=0)
    counts = jnp.sum(counts_ref[...], axis=(0, 2))
    mean = seg_sum / jnp.maximum(counts, 1.0)[:, None]
    h = (
        jnp.dot(u_ref[...], W1_ref[0:D, :], preferred_element_type=jnp.float32)
        + jnp.dot(mean, W1_ref[D:2 * D, :], preferred_element_type=jnp.float32)
        + b1_ref[...]
    )
    h = jnp.where(h >= 0, h, 0.01 * h)
    out_ref[...] = (
        jnp.dot(h, W2_ref[...], preferred_element_type=jnp.float32)
        + b2_ref[...]
    )


def _tc_finalize(sums, counts, u, W1, b1, W2, b2):
    return pl.pallas_call(
        _tc_body,
        out_shape=jax.ShapeDtypeStruct((NSEG, D), jnp.float32),
    )(sums, counts, u, W1, b1.reshape(1, -1), W2, b2.reshape(1, -1))


@jax.jit
def kernel(x, edge_index, edge_attr, u, batch, W1, b1, W2, b2):
    ids = batch.astype(jnp.int32)
    sums, counts = _sc_segment_partials(x, ids)
    return _tc_finalize(sums, counts, u, W1, b1, W2, b2)


# speedup vs baseline: 7.4960x; 2.4119x over previous
import jax
import jax.numpy as jnp
from jax import lax
from jax.experimental import pallas as pl
from jax.experimental.pallas import tpu as pltpu
from jax.experimental.pallas import tpu_sc as plsc

N_ROWS = 100000
D = 128
NSEG = 256
NC = 2
NS = 16
NW = NC * NS
CHUNK = 160
HALF = CHUNK // 2
NCHUNK = N_ROWS // CHUNK
MAXK = -(-NCHUNK // NW)
LANES = 16


def _sc_body(x_hbm, ids_hbm, ids2_hbm, sums_hbm, counts_hbm,
             xbufs, idbufs, id2bufs, cnt, zbuf, sems, shared_acc):
    cid = lax.axis_index("c")
    sid = lax.axis_index("s")
    wid = sid * NC + cid

    zeros16 = jnp.zeros((LANES,), jnp.float32)
    ones16 = jnp.ones((LANES,), jnp.float32)
    lane_iota = lax.iota(jnp.int32, LANES)
    my_acc = shared_acc.at[sid]

    for r in range(LANES):
        for c8 in range(D // LANES):
            zbuf[r, pl.ds(c8 * LANES, LANES)] = zeros16

    def zero_cnt(r, carry):
        cnt[r, :] = zeros16
        return carry

    lax.fori_loop(0, NSEG, zero_cnt, 0)
    for blk in range(NSEG // LANES):
        pltpu.sync_copy(zbuf, my_acc.at[pl.ds(blk * LANES, LANES), :])

    def start_loads(k, par):
        chunk_id = wid + NW * k

        @pl.when(chunk_id < NCHUNK)
        def _():
            base = chunk_id * CHUNK
            pltpu.async_copy(x_hbm.at[pl.ds(base, CHUNK), :], xbufs[par],
                             sems[par])
            pltpu.async_copy(ids_hbm.at[pl.ds(base, CHUNK)], idbufs[par],
                             sems[par])
            pltpu.async_copy(ids2_hbm.at[pl.ds(2 * chunk_id, 2), :],
                             id2bufs[par], sems[par])

    def wait_loads(k, par):
        chunk_id = wid + NW * k

        @pl.when(chunk_id < NCHUNK)
        def _():
            base = chunk_id * CHUNK
            pltpu.make_async_copy(x_hbm.at[pl.ds(base, CHUNK), :],
                                  xbufs[par], sems[par]).wait()
            pltpu.make_async_copy(ids_hbm.at[pl.ds(base, CHUNK)],
                                  idbufs[par], sems[par]).wait()
            pltpu.make_async_copy(ids2_hbm.at[pl.ds(2 * chunk_id, 2), :],
                                  id2bufs[par], sems[par]).wait()

    def process(k, par):
        chunk_id = wid + NW * k

        @pl.when(chunk_id < NCHUNK)
        def _():
            for j in range(CHUNK // HALF):
                pltpu.sync_copy(xbufs[par].at[pl.ds(j * HALF, HALF), :],
                                my_acc.at[id2bufs[par].at[j]], add=True)

            def cnt_body(g, carry):
                idv = idbufs[par][pl.ds(g * LANES, LANES)]
                plsc.addupdate_scatter(cnt, [idv, lane_iota], ones16)
                return carry

            lax.fori_loop(0, CHUNK // LANES, cnt_body, 0)

    start_loads(0, 0)

    def round_body(k2, carry):
        for par in range(2):
            k = k2 + par
            wait_loads(k, par)
            start_loads(k + 1, 1 - par)
            process(k, par)
        return carry

    lax.fori_loop(0, MAXK // 2, lambda i, c: round_body(i * 2, c), 0)

    pltpu.sync_copy(my_acc, sums_hbm.at[cid, sid])
    pltpu.sync_copy(cnt, counts_hbm.at[cid, sid])


def _sc_segment_partials(x, ids, ids2):
    mesh = plsc.VectorSubcoreMesh(core_axis_name="c", subcore_axis_name="s")
    f = pl.kernel(
        _sc_body,
        out_type=(
            jax.ShapeDtypeStruct((NC, NS, NSEG, D), jnp.float32),
            jax.ShapeDtypeStruct((NC, NS, NSEG, LANES), jnp.float32),
        ),
        mesh=mesh,
        scratch_types=(
            (pltpu.VMEM((CHUNK, D), jnp.float32),
             pltpu.VMEM((CHUNK, D), jnp.float32)),
            (pltpu.VMEM((CHUNK,), jnp.int32),
             pltpu.VMEM((CHUNK,), jnp.int32)),
            (pltpu.VMEM((CHUNK // HALF, HALF), jnp.int32),
             pltpu.VMEM((CHUNK // HALF, HALF), jnp.int32)),
            pltpu.VMEM((NSEG, LANES), jnp.float32),
            pltpu.VMEM((LANES, D), jnp.float32),
            (pltpu.SemaphoreType.DMA, pltpu.SemaphoreType.DMA),
            pltpu.VMEM_SHARED((NS, NSEG, D), jnp.float32),
        ),
        compiler_params=pltpu.CompilerParams(needs_layout_passes=False),
    )
    return f(x, ids, ids2)


def _tc_body(sums_ref, counts_ref, u_ref, W1_ref, b1_ref, W2_ref, b2_ref,
             out_ref):
    seg_sum = jnp.sum(sums_ref[...], axis=0)
    counts = jnp.sum(counts_ref[...], axis=(0, 2))
    mean = seg_sum / jnp.maximum(counts, 1.0)[:, None]
    h = (
        jnp.dot(u_ref[...], W1_ref[0:D, :], preferred_element_type=jnp.float32)
        + jnp.dot(mean, W1_ref[D:2 * D, :], preferred_element_type=jnp.float32)
        + b1_ref[...]
    )
    h = jnp.where(h >= 0, h, 0.01 * h)
    out_ref[...] = (
        jnp.dot(h, W2_ref[...], preferred_element_type=jnp.float32)
        + b2_ref[...]
    )


def _tc_finalize(sums, counts, u, W1, b1, W2, b2):
    return pl.pallas_call(
        _tc_body,
        out_shape=jax.ShapeDtypeStruct((NSEG, D), jnp.float32),
    )(sums, counts, u, W1, b1.reshape(1, -1), W2, b2.reshape(1, -1))


@jax.jit
def kernel(x, edge_index, edge_attr, u, batch, W1, b1, W2, b2):
    ids = batch.astype(jnp.int32)
    ids2 = ids.reshape(NCHUNK * (CHUNK // HALF), HALF)
    sums, counts = _sc_segment_partials(x, ids, ids2)
    sums = sums.reshape(NW, NSEG, D)
    counts = counts.reshape(NW, NSEG, LANES)
    return _tc_finalize(sums, counts, u, W1, b1, W2, b2)


# speedup vs baseline: 7.5408x; 1.0060x over previous
import jax
import jax.numpy as jnp
from jax import lax
from jax.experimental import pallas as pl
from jax.experimental.pallas import tpu as pltpu
from jax.experimental.pallas import tpu_sc as plsc

N_ROWS = 100000
D = 128
NSEG = 256
NC = 2
NS = 16
NW = NC * NS
CHUNK = 160
HALF = CHUNK // 2
NCHUNK = N_ROWS // CHUNK
MAXK = -(-NCHUNK // NW)
LANES = 16


def _sc_body(x_hbm, ids_hbm, ids2_hbm, sums_hbm, counts_hbm,
             xbufs, idbufs, id2bufs, cnt, zbuf, sems, ssems, shared_acc):
    cid = lax.axis_index("c")
    sid = lax.axis_index("s")
    wid = sid * NC + cid

    zeros16 = jnp.zeros((LANES,), jnp.float32)
    ones16 = jnp.ones((LANES,), jnp.float32)
    lane_iota = lax.iota(jnp.int32, LANES)
    my_acc = shared_acc.at[sid]

    for r in range(LANES):
        for c8 in range(D // LANES):
            zbuf[r, pl.ds(c8 * LANES, LANES)] = zeros16

    def zero_cnt(r, carry):
        cnt[r, :] = zeros16
        return carry

    lax.fori_loop(0, NSEG, zero_cnt, 0)
    for blk in range(NSEG // LANES):
        pltpu.sync_copy(zbuf, my_acc.at[pl.ds(blk * LANES, LANES), :])

    def start_loads(k, par):
        chunk_id = wid + NW * k

        @pl.when(chunk_id < NCHUNK)
        def _():
            base = chunk_id * CHUNK
            pltpu.async_copy(x_hbm.at[pl.ds(base, CHUNK), :], xbufs[par],
                             sems[par])
            pltpu.async_copy(ids_hbm.at[pl.ds(base, CHUNK)], idbufs[par],
                             sems[par])
            pltpu.async_copy(ids2_hbm.at[pl.ds(2 * chunk_id, 2), :],
                             id2bufs[par], sems[par])

    def wait_loads(k, par):
        chunk_id = wid + NW * k

        @pl.when(chunk_id < NCHUNK)
        def _():
            base = chunk_id * CHUNK
            pltpu.make_async_copy(x_hbm.at[pl.ds(base, CHUNK), :],
                                  xbufs[par], sems[par]).wait()
            pltpu.make_async_copy(ids_hbm.at[pl.ds(base, CHUNK)],
                                  idbufs[par], sems[par]).wait()
            pltpu.make_async_copy(ids2_hbm.at[pl.ds(2 * chunk_id, 2), :],
                                  id2bufs[par], sems[par]).wait()

    def fire_scatters(k, par):
        chunk_id = wid + NW * k

        @pl.when(chunk_id < NCHUNK)
        def _():
            for j in range(CHUNK // HALF):
                pltpu.async_copy(xbufs[par].at[pl.ds(j * HALF, HALF), :],
                                 my_acc.at[id2bufs[par].at[j]], ssems[par],
                                 add=True)

    def wait_scatters(k, par):
        chunk_id = wid + NW * k

        @pl.when((k >= 0) & (chunk_id < NCHUNK))
        def _():
            for j in range(CHUNK // HALF):
                pltpu.make_async_copy(
                    xbufs[par].at[pl.ds(j * HALF, HALF), :],
                    my_acc.at[id2bufs[par].at[j]], ssems[par]).wait()

    def do_counts(k, par):
        chunk_id = wid + NW * k

        @pl.when(chunk_id < NCHUNK)
        def _():
            def cnt_body(g, carry):
                idv = idbufs[par][pl.ds(g * LANES, LANES)]
                plsc.addupdate_scatter(cnt, [idv, lane_iota], ones16)
                return carry

            lax.fori_loop(0, CHUNK // LANES, cnt_body, 0)

    start_loads(0, 0)

    def round_body(k2, carry):
        for par in range(2):
            k = k2 + par
            wait_loads(k, par)
            fire_scatters(k, par)
            wait_scatters(k - 1, 1 - par)
            start_loads(k + 1, 1 - par)
            do_counts(k, par)
        return carry

    lax.fori_loop(0, MAXK // 2, lambda i, c: round_body(i * 2, c), 0)
    wait_scatters(MAXK - 1, (MAXK - 1) % 2)

    pltpu.sync_copy(my_acc, sums_hbm.at[cid, sid])
    pltpu.sync_copy(cnt, counts_hbm.at[cid, sid])


def _sc_segment_partials(x, ids, ids2):
    mesh = plsc.VectorSubcoreMesh(core_axis_name="c", subcore_axis_name="s")
    f = pl.kernel(
        _sc_body,
        out_type=(
            jax.ShapeDtypeStruct((NC, NS, NSEG, D), jnp.float32),
            jax.ShapeDtypeStruct((NC, NS, NSEG, LANES), jnp.float32),
        ),
        mesh=mesh,
        scratch_types=(
            (pltpu.VMEM((CHUNK, D), jnp.float32),
             pltpu.VMEM((CHUNK, D), jnp.float32)),
            (pltpu.VMEM((CHUNK,), jnp.int32),
             pltpu.VMEM((CHUNK,), jnp.int32)),
            (pltpu.VMEM((CHUNK // HALF, HALF), jnp.int32),
             pltpu.VMEM((CHUNK // HALF, HALF), jnp.int32)),
            pltpu.VMEM((NSEG, LANES), jnp.float32),
            pltpu.VMEM((LANES, D), jnp.float32),
            (pltpu.SemaphoreType.DMA, pltpu.SemaphoreType.DMA),
            (pltpu.SemaphoreType.DMA, pltpu.SemaphoreType.DMA),
            pltpu.VMEM_SHARED((NS, NSEG, D), jnp.float32),
        ),
        compiler_params=pltpu.CompilerParams(needs_layout_passes=False),
    )
    return f(x, ids, ids2)


def _tc_body(sums_ref, counts_ref, u_ref, W1_ref, b1_ref, W2_ref, b2_ref,
             out_ref):
    seg_sum = jnp.sum(sums_ref[...], axis=0)
    counts = jnp.sum(counts_ref[...], axis=(0, 2))
    mean = seg_sum / jnp.maximum(counts, 1.0)[:, None]
    h = (
        jnp.dot(u_ref[...], W1_ref[0:D, :], preferred_element_type=jnp.float32)
        + jnp.dot(mean, W1_ref[D:2 * D, :], preferred_element_type=jnp.float32)
        + b1_ref[...]
    )
    h = jnp.where(h >= 0, h, 0.01 * h)
    out_ref[...] = (
        jnp.dot(h, W2_ref[...], preferred_element_type=jnp.float32)
        + b2_ref[...]
    )


def _tc_finalize(sums, counts, u, W1, b1, W2, b2):
    return pl.pallas_call(
        _tc_body,
        out_shape=jax.ShapeDtypeStruct((NSEG, D), jnp.float32),
    )(sums, counts, u, W1, b1.reshape(1, -1), W2, b2.reshape(1, -1))


@jax.jit
def kernel(x, edge_index, edge_attr, u, batch, W1, b1, W2, b2):
    ids = batch.astype(jnp.int32)
    ids2 = ids.reshape(NCHUNK * (CHUNK // HALF), HALF)
    sums, counts = _sc_segment_partials(x, ids, ids2)
    sums = sums.reshape(NW, NSEG, D)
    counts = counts.reshape(NW, NSEG, LANES)
    return _tc_finalize(sums, counts, u, W1, b1, W2, b2)


# speedup vs baseline: 7.7890x; 1.0329x over previous
import jax
import jax.numpy as jnp
from jax import lax
from jax.experimental import pallas as pl
from jax.experimental.pallas import tpu as pltpu
from jax.experimental.pallas import tpu_sc as plsc

N_ROWS = 100000
D = 128
NSEG = 256
NC = 2
NS = 16
NW = NC * NS
CHUNK = 160
HALF = CHUNK // 2
NCHUNK = N_ROWS // CHUNK
MAXK = -(-NCHUNK // NW)
LANES = 16
NBUF = 3
ROUNDS = -(-MAXK // NBUF)


def _sc_body(x_hbm, ids2_hbm, sums_hbm, counts_hbm,
             xbufs, id2bufs, cnt, zbuf, sems, ssems, shared_acc):
    cid = lax.axis_index("c")
    sid = lax.axis_index("s")
    wid = sid * NC + cid

    zeros16 = jnp.zeros((LANES,), jnp.float32)
    ones16 = jnp.ones((LANES,), jnp.float32)
    lane_iota = lax.iota(jnp.int32, LANES)
    my_acc = shared_acc.at[sid]

    for r in range(LANES):
        for c8 in range(D // LANES):
            zbuf[r, pl.ds(c8 * LANES, LANES)] = zeros16

    def zero_cnt(r, carry):
        cnt[r, :] = zeros16
        return carry

    lax.fori_loop(0, NSEG, zero_cnt, 0)
    for blk in range(NSEG // LANES):
        pltpu.sync_copy(zbuf, my_acc.at[pl.ds(blk * LANES, LANES), :])

    def start_loads(k, par):
        chunk_id = wid + NW * k

        @pl.when(chunk_id < NCHUNK)
        def _():
            base = chunk_id * CHUNK
            pltpu.async_copy(x_hbm.at[pl.ds(base, CHUNK), :], xbufs[par],
                             sems[par])
            pltpu.async_copy(ids2_hbm.at[pl.ds(2 * chunk_id, 2), :],
                             id2bufs[par], sems[par])

    def wait_loads(k, par):
        chunk_id = wid + NW * k

        @pl.when(chunk_id < NCHUNK)
        def _():
            base = chunk_id * CHUNK
            pltpu.make_async_copy(x_hbm.at[pl.ds(base, CHUNK), :],
                                  xbufs[par], sems[par]).wait()
            pltpu.make_async_copy(ids2_hbm.at[pl.ds(2 * chunk_id, 2), :],
                                  id2bufs[par], sems[par]).wait()

    def fire_scatters(k, par):
        chunk_id = wid + NW * k

        @pl.when(chunk_id < NCHUNK)
        def _():
            for j in range(CHUNK // HALF):
                pltpu.async_copy(xbufs[par].at[pl.ds(j * HALF, HALF), :],
                                 my_acc.at[id2bufs[par].at[j]], ssems[par],
                                 add=True)

    def wait_scatters(k, par):
        chunk_id = wid + NW * k

        @pl.when((k >= 0) & (chunk_id < NCHUNK))
        def _():
            for j in range(CHUNK // HALF):
                pltpu.make_async_copy(
                    xbufs[par].at[pl.ds(j * HALF, HALF), :],
                    my_acc.at[id2bufs[par].at[j]], ssems[par]).wait()

    def do_counts(k, par):
        chunk_id = wid + NW * k

        @pl.when(chunk_id < NCHUNK)
        def _():
            for j in range(CHUNK // HALF):
                def cnt_body(g, carry, j=j):
                    idv = id2bufs[par][j, pl.ds(g * LANES, LANES)]
                    plsc.addupdate_scatter(cnt, [idv, lane_iota], ones16)
                    return carry

                lax.fori_loop(0, HALF // LANES, cnt_body, 0)

    for i in range(NBUF - 1):
        start_loads(i, i)

    def round_body(k4, carry):
        for par in range(NBUF):
            k = k4 + par
            wait_loads(k, par)
            fire_scatters(k, par)
            wait_scatters(k - 1, (par - 1) % NBUF)
            start_loads(k + NBUF - 1, (par - 1) % NBUF)
            do_counts(k, par)
        return carry

    lax.fori_loop(0, ROUNDS, lambda i, c: round_body(i * NBUF, c), 0)
    wait_scatters(ROUNDS * NBUF - 1, NBUF - 1)

    pltpu.sync_copy(my_acc, sums_hbm.at[cid, sid])
    pltpu.sync_copy(cnt, counts_hbm.at[cid, sid])


def _sc_segment_partials(x, ids2):
    mesh = plsc.VectorSubcoreMesh(core_axis_name="c", subcore_axis_name="s")
    f = pl.kernel(
        _sc_body,
        out_type=(
            jax.ShapeDtypeStruct((NC, NS, NSEG, D), jnp.float32),
            jax.ShapeDtypeStruct((NC, NS, NSEG, LANES), jnp.float32),
        ),
        mesh=mesh,
        scratch_types=(
            tuple(pltpu.VMEM((CHUNK, D), jnp.float32) for _ in range(NBUF)),
            tuple(pltpu.VMEM((CHUNK // HALF, HALF), jnp.int32)
                  for _ in range(NBUF)),
            pltpu.VMEM((NSEG, LANES), jnp.float32),
            pltpu.VMEM((LANES, D), jnp.float32),
            tuple(pltpu.SemaphoreType.DMA for _ in range(NBUF)),
            tuple(pltpu.SemaphoreType.DMA for _ in range(NBUF)),
            pltpu.VMEM_SHARED((NS, NSEG, D), jnp.float32),
        ),
        compiler_params=pltpu.CompilerParams(needs_layout_passes=False),
    )
    return f(x, ids2)


def _tc_body(sums_ref, counts_ref, u_ref, W1_ref, b1_ref, W2_ref, b2_ref,
             out_ref):
    seg_sum = jnp.sum(sums_ref[...], axis=0)
    counts = jnp.sum(counts_ref[...], axis=(0, 2))
    mean = seg_sum / jnp.maximum(counts, 1.0)[:, None]
    h = (
        jnp.dot(u_ref[...], W1_ref[0:D, :], preferred_element_type=jnp.float32)
        + jnp.dot(mean, W1_ref[D:2 * D, :], preferred_element_type=jnp.float32)
        + b1_ref[...]
    )
    h = jnp.where(h >= 0, h, 0.01 * h)
    out_ref[...] = (
        jnp.dot(h, W2_ref[...], preferred_element_type=jnp.float32)
        + b2_ref[...]
    )


def _tc_finalize(sums, counts, u, W1, b1, W2, b2):
    return pl.pallas_call(
        _tc_body,
        out_shape=jax.ShapeDtypeStruct((NSEG, D), jnp.float32),
    )(sums, counts, u, W1, b1.reshape(1, -1), W2, b2.reshape(1, -1))


@jax.jit
def kernel(x, edge_index, edge_attr, u, batch, W1, b1, W2, b2):
    ids = batch.astype(jnp.int32)
    ids2 = ids.reshape(NCHUNK * (CHUNK // HALF), HALF)
    sums, counts = _sc_segment_partials(x, ids2)
    sums = sums.reshape(NW, NSEG, D)
    counts = counts.reshape(NW, NSEG, LANES)
    return _tc_finalize(sums, counts, u, W1, b1, W2, b2)


# speedup vs baseline: 9.4986x; 1.2195x over previous
import jax
import jax.numpy as jnp
from jax import lax
from jax.experimental import pallas as pl
from jax.experimental.pallas import tpu as pltpu
from jax.experimental.pallas import tpu_sc as plsc

N_ROWS = 100000
D = 128
NSEG = 256
NC = 2
NS = 16
NW = NC * NS
CHUNK = 160
HALF = CHUNK // 2
LANES = 16
NBUF = 3

SC_ROWS = 60000
TC_ROWS = N_ROWS - SC_ROWS
TC_BLK = 2000
TC_NBLK = TC_ROWS // TC_BLK

NCHUNK = SC_ROWS // CHUNK
MAXK = -(-NCHUNK // NW)
ROUNDS = -(-MAXK // NBUF)


def _sc_body(x_hbm, ids2_hbm, sums_hbm, counts_hbm,
             xbufs, id2bufs, cnt, zbuf, sems, ssems, shared_acc):
    cid = lax.axis_index("c")
    sid = lax.axis_index("s")
    wid = sid * NC + cid

    zeros16 = jnp.zeros((LANES,), jnp.float32)
    ones16 = jnp.ones((LANES,), jnp.float32)
    lane_iota = lax.iota(jnp.int32, LANES)
    my_acc = shared_acc.at[sid]

    for r in range(LANES):
        for c8 in range(D // LANES):
            zbuf[r, pl.ds(c8 * LANES, LANES)] = zeros16

    def zero_cnt(r, carry):
        cnt[r, :] = zeros16
        return carry

    lax.fori_loop(0, NSEG, zero_cnt, 0)
    for blk in range(NSEG // LANES):
        pltpu.sync_copy(zbuf, my_acc.at[pl.ds(blk * LANES, LANES), :])

    def start_loads(k, par):
        chunk_id = wid + NW * k

        @pl.when(chunk_id < NCHUNK)
        def _():
            base = chunk_id * CHUNK
            pltpu.async_copy(x_hbm.at[pl.ds(base, CHUNK), :], xbufs[par],
                             sems[par])
            pltpu.async_copy(ids2_hbm.at[pl.ds(2 * chunk_id, 2), :],
                             id2bufs[par], sems[par])

    def wait_loads(k, par):
        chunk_id = wid + NW * k

        @pl.when(chunk_id < NCHUNK)
        def _():
            base = chunk_id * CHUNK
            pltpu.make_async_copy(x_hbm.at[pl.ds(base, CHUNK), :],
                                  xbufs[par], sems[par]).wait()
            pltpu.make_async_copy(ids2_hbm.at[pl.ds(2 * chunk_id, 2), :],
                                  id2bufs[par], sems[par]).wait()

    def fire_scatters(k, par):
        chunk_id = wid + NW * k

        @pl.when(chunk_id < NCHUNK)
        def _():
            for j in range(CHUNK // HALF):
                pltpu.async_copy(xbufs[par].at[pl.ds(j * HALF, HALF), :],
                                 my_acc.at[id2bufs[par].at[j]], ssems[par],
                                 add=True)

    def wait_scatters(k, par):
        chunk_id = wid + NW * k

        @pl.when((k >= 0) & (chunk_id < NCHUNK))
        def _():
            for j in range(CHUNK // HALF):
                pltpu.make_async_copy(
                    xbufs[par].at[pl.ds(j * HALF, HALF), :],
                    my_acc.at[id2bufs[par].at[j]], ssems[par]).wait()

    def do_counts(k, par):
        chunk_id = wid + NW * k

        @pl.when(chunk_id < NCHUNK)
        def _():
            for j in range(CHUNK // HALF):
                def cnt_body(g, carry, j=j):
                    idv = id2bufs[par][j, pl.ds(g * LANES, LANES)]
                    plsc.addupdate_scatter(cnt, [idv, lane_iota], ones16)
                    return carry

                lax.fori_loop(0, HALF // LANES, cnt_body, 0)

    for i in range(NBUF - 1):
        start_loads(i, i)

    def round_body(k0, carry):
        for par in range(NBUF):
            k = k0 + par
            wait_loads(k, par)
            fire_scatters(k, par)
            wait_scatters(k - 1, (par - 1) % NBUF)
            start_loads(k + NBUF - 1, (par - 1) % NBUF)
            do_counts(k, par)
        return carry

    lax.fori_loop(0, ROUNDS, lambda i, c: round_body(i * NBUF, c), 0)
    wait_scatters(ROUNDS * NBUF - 1, NBUF - 1)

    pltpu.sync_copy(my_acc, sums_hbm.at[cid, sid])
    pltpu.sync_copy(cnt, counts_hbm.at[cid, sid])


def _sc_segment_partials(x, ids2):
    mesh = plsc.VectorSubcoreMesh(core_axis_name="c", subcore_axis_name="s")
    f = pl.kernel(
        _sc_body,
        out_type=(
            jax.ShapeDtypeStruct((NC, NS, NSEG, D), jnp.float32),
            jax.ShapeDtypeStruct((NC, NS, NSEG, LANES), jnp.float32),
        ),
        mesh=mesh,
        scratch_types=(
            tuple(pltpu.VMEM((CHUNK, D), jnp.float32) for _ in range(NBUF)),
            tuple(pltpu.VMEM((CHUNK // HALF, HALF), jnp.int32)
                  for _ in range(NBUF)),
            pltpu.VMEM((NSEG, LANES), jnp.float32),
            pltpu.VMEM((LANES, D), jnp.float32),
            tuple(pltpu.SemaphoreType.DMA for _ in range(NBUF)),
            tuple(pltpu.SemaphoreType.DMA for _ in range(NBUF)),
            pltpu.VMEM_SHARED((NS, NSEG, D), jnp.float32),
        ),
        compiler_params=pltpu.CompilerParams(needs_layout_passes=False),
    )
    return f(x, ids2)


def _tc_seg_body(ids_ref, x_ref, sums_ref, cnt_ref, acc, cnt):
    i = pl.program_id(0)

    @pl.when(i == 0)
    def _():
        acc[...] = jnp.zeros_like(acc)
        cnt[...] = jnp.zeros_like(cnt)

    ids = ids_ref[0, 0, :]
    oh = (ids[:, None]
          == lax.broadcasted_iota(jnp.int32, (TC_BLK, NSEG), 1)
          ).astype(jnp.float32)
    acc[...] += lax.dot_general(oh, x_ref[...], (((0,), (0,)), ((), ())),
                                preferred_element_type=jnp.float32)
    cnt[...] += jnp.sum(oh, axis=0)[None, :]

    @pl.when(i == TC_NBLK - 1)
    def _():
        sums_ref[...] = acc[...]
        cnt_ref[...] = cnt[...]


def _tc_segment_partials(x, ids3):
    base_blk = SC_ROWS // TC_BLK
    return pl.pallas_call(
        _tc_seg_body,
        grid=(TC_NBLK,),
        in_specs=[
            pl.BlockSpec((1, 1, TC_BLK), lambda i: (i, 0, 0)),
            pl.BlockSpec((TC_BLK, D), lambda i: (i + base_blk, 0)),
        ],
        out_specs=[
            pl.BlockSpec((NSEG, D), lambda i: (0, 0)),
            pl.BlockSpec((1, NSEG), lambda i: (0, 0)),
        ],
        out_shape=(
            jax.ShapeDtypeStruct((NSEG, D), jnp.float32),
            jax.ShapeDtypeStruct((1, NSEG), jnp.float32),
        ),
        scratch_shapes=[
            pltpu.VMEM((NSEG, D), jnp.float32),
            pltpu.VMEM((1, NSEG), jnp.float32),
        ],
    )(ids3, x)


def _tc_body(sums_ref, counts_ref, tsum_ref, tcnt_ref,
             u_ref, W1_ref, b1_ref, W2_ref, b2_ref, out_ref):
    seg_sum = jnp.sum(sums_ref[...], axis=0) + tsum_ref[...]
    counts = jnp.sum(counts_ref[...], axis=(0, 2)) + tcnt_ref[0, :]
    mean = seg_sum / jnp.maximum(counts, 1.0)[:, None]
    h = (
        jnp.dot(u_ref[...], W1_ref[0:D, :], preferred_element_type=jnp.float32)
        + jnp.dot(mean, W1_ref[D:2 * D, :], preferred_element_type=jnp.float32)
        + b1_ref[...]
    )
    h = jnp.where(h >= 0, h, 0.01 * h)
    out_ref[...] = (
        jnp.dot(h, W2_ref[...], preferred_element_type=jnp.float32)
        + b2_ref[...]
    )


def _tc_finalize(sums, counts, tsum, tcnt, u, W1, b1, W2, b2):
    return pl.pallas_call(
        _tc_body,
        out_shape=jax.ShapeDtypeStruct((NSEG, D), jnp.float32),
    )(sums, counts, tsum, tcnt, u, W1, b1.reshape(1, -1), W2,
      b2.reshape(1, -1))


@jax.jit
def kernel(x, edge_index, edge_attr, u, batch, W1, b1, W2, b2):
    ids = batch.astype(jnp.int32)
    ids2 = ids[:SC_ROWS].reshape(NCHUNK * (CHUNK // HALF), HALF)
    ids3 = ids[SC_ROWS:].reshape(TC_NBLK, 1, TC_BLK)
    sums, counts = _sc_segment_partials(x, ids2)
    tsum, tcnt = _tc_segment_partials(x, ids3)
    sums = sums.reshape(NW, NSEG, D)
    counts = counts.reshape(NW, NSEG, LANES)
    return _tc_finalize(sums, counts, tsum, tcnt, u, W1, b1, W2, b2)
